# baseline (speedup 1.0000x reference)
import functools

import jax
import jax.numpy as jnp
from jax import lax
from jax.experimental import pallas as pl
from jax.experimental.pallas import tpu as pltpu
from jax.experimental.pallas import tpu_sc as plsc

N = 10000
E = 160000
DF = 256
DE = 16

NUM_CORES = 2
NUM_SUBCORES = 16
EB = E // 128
EB_HALF = EB // NUM_CORES
CHUNK_EB = 125
N_CHUNKS = EB_HALF // CHUNK_EB
CHUNK_EDGES = CHUNK_EB * 128
NPAD = 10240
UNROLL = 8


def _sc_segment_sum(recv3, q, zeros_init):
    mesh = plsc.VectorSubcoreMesh(core_axis_name="c", subcore_axis_name="s")

    @functools.partial(
        pl.kernel,
        mesh=mesh,
        compiler_params=pltpu.CompilerParams(
            use_tc_tiling_on_sc=False, needs_layout_passes=False
        ),
        out_type=jax.ShapeDtypeStruct((NUM_CORES, DE, NPAD), jnp.float32),
        scratch_types=[
            pltpu.VMEM((2, CHUNK_EB, 128), jnp.int32),
            pltpu.VMEM((2, CHUNK_EB, 128), jnp.float32),
            pltpu.VMEM((NPAD,), jnp.float32),
            pltpu.SemaphoreType.DMA,
        ],
    )
    def sc_kernel(recv_hbm, q_hbm, zeros_hbm, out_hbm, idx_v, val_v, plane, sem):
        c = lax.axis_index("c")
        s = lax.axis_index("s")
        fb = s // 8
        fi = s % 8
        eb0 = c * EB_HALF

        pltpu.sync_copy(zeros_hbm, plane)

        def start_chunk(j, buf):
            base = eb0 + j * CHUNK_EB
            pltpu.make_async_copy(
                recv_hbm.at[pl.ds(base, CHUNK_EB), 1, :], idx_v.at[buf], sem
            ).start()
            pltpu.make_async_copy(
                q_hbm.at[fb, pl.ds(base, CHUNK_EB), fi, :], val_v.at[buf], sem
            ).start()

        def wait_chunk(buf):
            pltpu.make_async_copy(
                recv_hbm.at[pl.ds(0, CHUNK_EB), 1, :], idx_v.at[buf], sem
            ).wait()
            pltpu.make_async_copy(
                q_hbm.at[0, pl.ds(0, CHUNK_EB), 0, :], val_v.at[buf], sem
            ).wait()

        start_chunk(0, 0)

        for j in range(N_CHUNKS):
            buf = j % 2
            if j + 1 < N_CHUNKS:
                start_chunk(j + 1, 1 - buf)
            wait_chunk(buf)

            @plsc.parallel_loop(0, CHUNK_EB, unroll=4)
            def _(t):
                for u in range(UNROLL):
                    idx16 = idx_v[buf, t, pl.ds(u * 16, 16)]
                    val16 = val_v[buf, t, pl.ds(u * 16, 16)]
                    plsc.addupdate_scatter(plane, [idx16], val16)

        pltpu.sync_copy(plane, out_hbm.at[c, s])

    return sc_kernel(recv3, q, zeros_init)


ROW_BLOCK = 2000
ROW_BLOCK2 = 2560


def _mm1_body(x_ref, w1_ref, b_ref, o_ref):
    o_ref[...] = (
        jnp.dot(x_ref[...], w1_ref[...], preferred_element_type=jnp.float32)
        + b_ref[...]
    )


def _tc_mm1(x, W1, b2d):
    return pl.pallas_call(
        _mm1_body,
        grid=(N // ROW_BLOCK,),
        in_specs=[
            pl.BlockSpec((ROW_BLOCK, DF), lambda i: (i, 0)),
            pl.BlockSpec((DF, DF), lambda i: (0, 0)),
            pl.BlockSpec((1, DF), lambda i: (0, 0)),
        ],
        out_specs=pl.BlockSpec((ROW_BLOCK, DF), lambda i: (i, 0)),
        out_shape=jax.ShapeDtypeStruct((N, DF), jnp.float32),
    )(x, W1, b2d)


def _mm2_body(o1_ref, p_ref, w2_ref, o_ref):
    aggr_t = p_ref[0] + p_ref[1]
    upd = jax.lax.dot_general(
        aggr_t,
        w2_ref[...],
        dimension_numbers=(((0,), (0,)), ((), ())),
        preferred_element_type=jnp.float32,
    )
    o_ref[...] = o1_ref[...] + upd


def _tc_mm2(out1, partials, W2):
    grid = (N + ROW_BLOCK2 - 1) // ROW_BLOCK2
    return pl.pallas_call(
        _mm2_body,
        grid=(grid,),
        in_specs=[
            pl.BlockSpec((ROW_BLOCK2, DF), lambda i: (i, 0)),
            pl.BlockSpec((NUM_CORES, DE, ROW_BLOCK2), lambda i: (0, 0, i)),
            pl.BlockSpec((DE, DF), lambda i: (0, 0)),
        ],
        out_specs=pl.BlockSpec((ROW_BLOCK2, DF), lambda i: (i, 0)),
        out_shape=jax.ShapeDtypeStruct((N, DF), jnp.float32),
    )(out1, partials, W2)


def kernel(x, edge_index, edge_attr, pos, W, b):
    recv3 = edge_index.reshape(2, EB, 128).transpose(1, 0, 2)
    q = edge_attr.reshape(EB, 128, 2, 8).transpose(2, 0, 3, 1)
    zeros_init = jnp.zeros((NPAD,), jnp.float32)
    partials = _sc_segment_sum(recv3, q, zeros_init)
    W1 = W[:DF]
    W2 = W[DF:]
    b2d = b.reshape(1, DF)
    out1 = _tc_mm1(x, W1, b2d)
    return _tc_mm2(out1, partials, W2)


# speedup vs baseline: 1.0200x; 1.0200x over previous
import functools

import jax
import jax.numpy as jnp
from jax import lax
from jax.experimental import pallas as pl
from jax.experimental.pallas import tpu as pltpu
from jax.experimental.pallas import tpu_sc as plsc

N = 10000
E = 160000
DF = 256
DE = 16

NUM_CORES = 2
NUM_SUBCORES = 16
EB = E // 128
EB_HALF = EB // NUM_CORES
CHUNK_EB = 125
N_CHUNKS = EB_HALF // CHUNK_EB
CHUNK_EDGES = CHUNK_EB * 128
NPAD = 10240
UNROLL = 8


def _sc_segment_sum(recv3, q, zeros_init):
    mesh = plsc.VectorSubcoreMesh(core_axis_name="c", subcore_axis_name="s")

    @functools.partial(
        pl.kernel,
        mesh=mesh,
        compiler_params=pltpu.CompilerParams(
            use_tc_tiling_on_sc=False, needs_layout_passes=False
        ),
        out_type=jax.ShapeDtypeStruct((NUM_CORES, DE, NPAD), jnp.float32),
        scratch_types=[
            pltpu.VMEM((2, CHUNK_EB, 128), jnp.int32),
            pltpu.VMEM((2, CHUNK_EB, 128), jnp.float32),
            pltpu.VMEM((NPAD,), jnp.float32),
            pltpu.SemaphoreType.DMA,
        ],
    )
    def sc_kernel(recv_hbm, q_hbm, zeros_hbm, out_hbm, idx_v, val_v, plane, sem):
        c = lax.axis_index("c")
        s = lax.axis_index("s")
        fb = s // 8
        fi = s % 8
        eb0 = c * EB_HALF

        def start_chunk(j, buf):
            base = eb0 + j * CHUNK_EB
            pltpu.make_async_copy(
                recv_hbm.at[pl.ds(base, CHUNK_EB), 1, :], idx_v.at[buf], sem
            ).start()
            pltpu.make_async_copy(
                q_hbm.at[fb, pl.ds(base, CHUNK_EB), fi, :], val_v.at[buf], sem
            ).start()

        def wait_chunk(buf):
            pltpu.make_async_copy(
                recv_hbm.at[pl.ds(0, CHUNK_EB), 1, :], idx_v.at[buf], sem
            ).wait()
            pltpu.make_async_copy(
                q_hbm.at[0, pl.ds(0, CHUNK_EB), 0, :], val_v.at[buf], sem
            ).wait()

        start_chunk(0, 0)

        pltpu.sync_copy(zeros_hbm, plane)

        for j in range(N_CHUNKS):
            buf = j % 2
            if j + 1 < N_CHUNKS:
                start_chunk(j + 1, 1 - buf)
            wait_chunk(buf)

            @plsc.parallel_loop(0, CHUNK_EB, unroll=2)
            def _(t):
                for u in range(UNROLL):
                    idx16 = idx_v[buf, t, pl.ds(u * 16, 16)]
                    val16 = val_v[buf, t, pl.ds(u * 16, 16)]
                    plsc.addupdate_scatter(plane, [idx16], val16)

        pltpu.sync_copy(plane, out_hbm.at[c, s])

    return sc_kernel(recv3, q, zeros_init)


ROW_BLOCK = 2000
ROW_BLOCK2 = 2560


def _mm1_body(x_ref, w1_ref, b_ref, o_ref):
    o_ref[...] = (
        jnp.dot(x_ref[...], w1_ref[...], preferred_element_type=jnp.float32)
        + b_ref[...]
    )


def _tc_mm1(x, W1, b2d):
    return pl.pallas_call(
        _mm1_body,
        grid=(N // ROW_BLOCK,),
        in_specs=[
            pl.BlockSpec((ROW_BLOCK, DF), lambda i: (i, 0)),
            pl.BlockSpec((DF, DF), lambda i: (0, 0)),
            pl.BlockSpec((1, DF), lambda i: (0, 0)),
        ],
        out_specs=pl.BlockSpec((ROW_BLOCK, DF), lambda i: (i, 0)),
        out_shape=jax.ShapeDtypeStruct((N, DF), jnp.float32),
    )(x, W1, b2d)


def _mm2_body(o1_ref, p_ref, w2_ref, o_ref):
    aggr_t = p_ref[0] + p_ref[1]
    upd = jax.lax.dot_general(
        aggr_t,
        w2_ref[...],
        dimension_numbers=(((0,), (0,)), ((), ())),
        preferred_element_type=jnp.float32,
    )
    o_ref[...] = o1_ref[...] + upd


def _tc_mm2(out1, partials, W2):
    grid = (N + ROW_BLOCK2 - 1) // ROW_BLOCK2
    return pl.pallas_call(
        _mm2_body,
        grid=(grid,),
        in_specs=[
            pl.BlockSpec((ROW_BLOCK2, DF), lambda i: (i, 0)),
            pl.BlockSpec((NUM_CORES, DE, ROW_BLOCK2), lambda i: (0, 0, i)),
            pl.BlockSpec((DE, DF), lambda i: (0, 0)),
        ],
        out_specs=pl.BlockSpec((ROW_BLOCK2, DF), lambda i: (i, 0)),
        out_shape=jax.ShapeDtypeStruct((N, DF), jnp.float32),
    )(out1, partials, W2)


def kernel(x, edge_index, edge_attr, pos, W, b):
    recv3 = edge_index.reshape(2, EB, 128).transpose(1, 0, 2)
    q = edge_attr.reshape(EB, 128, 2, 8).transpose(2, 0, 3, 1)
    zeros_init = jnp.zeros((NPAD,), jnp.float32)
    partials = _sc_segment_sum(recv3, q, zeros_init)
    W1 = W[:DF]
    W2 = W[DF:]
    b2d = b.reshape(1, DF)
    out1 = _tc_mm1(x, W1, b2d)
    return _tc_mm2(out1, partials, W2)
